# baseline (speedup 1.0000x reference)
import functools

import jax
import jax.numpy as jnp
from jax import lax
from jax.experimental import pallas as pl
from jax.experimental.pallas import tpu as pltpu
from jax.experimental.pallas import tpu_sc as plsc

_FIELD0 = 1000000
_B = 16384
_D = 16
_NC, _NS = 2, 16
_NW = _NC * _NS
_BPW = _B // _NW
_CH = 128
_NCH = _BPW // _CH
_NG = _BPW // _D


def _body(x0_hbm, x1_hbm, table_hbm, out_hbm,
          idx0_v, idx1_v, rows0_v, rows1_v, prod_v, out_v, sem):
  wid = lax.axis_index("s") * _NC + lax.axis_index("c")
  base = wid * _BPW

  for j in range(_NCH):
    pltpu.sync_copy(x0_hbm.at[pl.ds(base + j * _CH, _CH)], idx0_v.at[j])
    pltpu.sync_copy(x1_hbm.at[pl.ds(base + j * _CH, _CH)], idx1_v.at[j])
  for j in range(_NCH):
    for k in range(_CH // _D):
      sl = pl.ds(k * _D, _D)
      idx1_v[j, sl] = idx1_v[j, sl] + _FIELD0

  copies = []
  for j in range(_NCH):
    dst = rows0_v.at[pl.ds(j * _CH, _CH), :]
    copies.append(pltpu.async_copy(table_hbm.at[idx0_v.at[j]], dst, sem))
    dst = rows1_v.at[pl.ds(j * _CH, _CH), :]
    copies.append(pltpu.async_copy(table_hbm.at[idx1_v.at[j]], dst, sem))
  for c in copies:
    c.wait()

  lane16 = lax.iota(jnp.int32, _D) * _D

  def group(g, carry):
    for k in range(_D):
      r = g * _D + k
      prod_v[pl.ds(k * _D, _D)] = rows0_v[r, :] * rows1_v[r, :]
    acc = jnp.zeros((_D,), jnp.float32)
    for l in range(_D):
      acc = acc + plsc.load_gather(prod_v, [lane16 + l])
    e = jnp.exp(-jnp.abs(acc))
    out_v[pl.ds(g * _D, _D)] = jnp.where(acc >= 0.0, 1.0 / (1.0 + e),
                                         e / (1.0 + e))
    return carry

  lax.fori_loop(0, _NG, group, 0)

  pltpu.sync_copy(out_v, out_hbm.at[pl.ds(base, _BPW)])


@functools.partial(jax.jit, donate_argnums=())
def _run(x0, x1, table):
  mesh = plsc.VectorSubcoreMesh(core_axis_name="c", subcore_axis_name="s",
                                num_cores=_NC, num_subcores=_NS)
  return pl.kernel(
      _body,
      out_type=jax.ShapeDtypeStruct((_B,), jnp.float32),
      mesh=mesh,
      compiler_params=pltpu.CompilerParams(needs_layout_passes=False,
                                           use_tc_tiling_on_sc=False),
      scratch_types=[
          pltpu.VMEM((_NCH, _CH), jnp.int32),
          pltpu.VMEM((_NCH, _CH), jnp.int32),
          pltpu.VMEM((_BPW, _D), jnp.float32),
          pltpu.VMEM((_BPW, _D), jnp.float32),
          pltpu.VMEM((_D * _D,), jnp.float32),
          pltpu.VMEM((_BPW,), jnp.float32),
          pltpu.SemaphoreType.DMA,
      ],
  )(x0, x1, table)


def kernel(x, table):
  x0 = jnp.asarray(x[:, 0], jnp.int32)
  x1 = jnp.asarray(x[:, 1], jnp.int32)
  return _run(x0, x1, table).reshape(_B, 1)
